# Initial kernel scaffold; baseline (speedup 1.0000x reference)
import jax
import jax.numpy as jnp
from jax.experimental import pallas as pl


def kernel(x, pos, Wh0, bh0, Wx0, bx0, Wh1, bh1, Wx1, bx1, Wh2, bh2, Wx2, bx2, Wfc, bfc, Wt, bt):
    raise NotImplementedError("write your pallas kernel here")


# speedup vs baseline: 1.9010x; 1.9010x over previous
import functools

import jax
import jax.numpy as jnp
from jax import lax
from jax.experimental import pallas as pl
from jax.experimental.pallas import tpu as pltpu
from jax.experimental.pallas import tpu_sc as plsc

NN = 10000
K = 6
D = 128
RB = 256
CT = 512
LB = 512
NP = 10240
PADV = 1.0e18
ROWW = 144
NW = 32
NPW = NP // NW
NB = 64
NT = NPW // NB

_BIG_F = 3.0e38
_BIG_I = (1 << 30)


def _top6(vals, ids):
    out_v, out_i = [], []
    for _ in range(K):
        m = jnp.min(vals, axis=1, keepdims=True)
        idx = jnp.min(jnp.where(vals == m, ids, _BIG_I), axis=1,
                      keepdims=True)
        out_v.append(m)
        out_i.append(idx)
        vals = jnp.where(ids == idx, _BIG_F, vals)
    return out_v, out_i


def _knn_body(pos_r_ref, pos_t_ref, col_ref, bv_ref, bi_ref):
    i = pl.program_id(0)
    c = pl.program_id(1)

    @pl.when(c == 0)
    def _():
        bv_ref[:] = jnp.full((RB, 8), _BIG_F, jnp.float32)
        bi_ref[:] = jnp.zeros((RB, 8), jnp.int32)

    pr = pos_r_ref[:]
    pt = pos_t_ref[:]
    dot = jnp.dot(pr, pt, preferred_element_type=jnp.float32)
    sqr = (pr[:, 0:1] * pr[:, 0:1] + pr[:, 1:2] * pr[:, 1:2]) \
        + pr[:, 2:3] * pr[:, 2:3]
    sqc = (pt[0:1, :] * pt[0:1, :] + pt[1:2, :] * pt[1:2, :]) \
        + pt[2:3, :] * pt[2:3, :]
    d2 = (sqr + sqc) - 2.0 * dot
    colids = lax.broadcasted_iota(jnp.int32, (RB, CT), 1) + c * CT
    row_ids = lax.broadcasted_iota(jnp.int32, (RB, 1), 0) + i * RB
    d2 = jnp.where(colids == row_ids, _BIG_F, d2)
    tv, ti = _top6(d2, colids)

    lv, li = [], []
    for k in range(K):
        a_v = bv_ref[:, k:k + 1]
        a_i = bi_ref[:, k:k + 1]
        b_v = tv[K - 1 - k]
        b_i = ti[K - 1 - k]
        sel = a_v <= b_v
        lv.append(jnp.where(sel, a_v, b_v))
        li.append(jnp.where(sel, a_i, b_i))
    for r in range(K):
        for p, q in ((0, 1), (2, 3), (4, 5)) if r % 2 == 0 else \
                ((1, 2), (3, 4)):
            sel = lv[p] <= lv[q]
            pv = jnp.where(sel, lv[p], lv[q])
            qv = jnp.where(sel, lv[q], lv[p])
            pi = jnp.where(sel, li[p], li[q])
            qi = jnp.where(sel, li[q], li[p])
            lv[p], lv[q], li[p], li[q] = pv, qv, pi, qi
    for k in range(K):
        bv_ref[:, k:k + 1] = lv[k]
        bi_ref[:, k:k + 1] = li[k]

    @pl.when(c == NP // CT - 1)
    def _():
        for k in range(K):
            col_ref[:, k:k + 1] = li[k]


def _knn(pos_p, pos_t):
    return pl.pallas_call(
        _knn_body,
        grid=(NP // RB, NP // CT),
        in_specs=[
            pl.BlockSpec((RB, 3), lambda i, c: (i, 0)),
            pl.BlockSpec((3, CT), lambda i, c: (0, c)),
        ],
        out_specs=pl.BlockSpec((RB, K), lambda i, c: (i, 0)),
        out_shape=jax.ShapeDtypeStruct((NP, K), jnp.int32),
        scratch_shapes=[
            pltpu.VMEM((RB, 8), jnp.float32),
            pltpu.VMEM((RB, 8), jnp.int32),
        ],
        compiler_params=pltpu.CompilerParams(
            dimension_semantics=("parallel", "arbitrary")),
    )(pos_p, pos_t)


def _layer_body(hp_ref, xxp_ref, sp_ref, wh_ref, bh_ref, wxt_ref, bx_ref,
                mq_ref, hn_ref, xxn_ref):
    h = hp_ref[:] + sp_ref[:, 0:D]
    sw = sp_ref[:, 131:132]
    sxw = sp_ref[:, 128:131]
    xyz = xxp_ref[:, 0:3]
    xyz = xyz + xyz * sw - sxw
    m = jnp.maximum(
        jnp.dot(h, wh_ref[:], preferred_element_type=jnp.float32)
        + bh_ref[:], 0.0)
    w = jnp.maximum(
        jnp.sum(h * wxt_ref[:], axis=1, keepdims=True) + bx_ref[:], 0.0)
    mq_ref[:, 0:D] = m
    mq_ref[:, 128:131] = xyz * w
    mq_ref[:, 131:132] = w
    mq_ref[:, 132:ROWW] = jnp.zeros((LB, ROWW - 132), jnp.float32)
    hn_ref[:] = h
    xxn_ref[:, 0:3] = xyz
    xxn_ref[:, 3:4] = jnp.zeros((LB, 1), jnp.float32)


def _layer(h, xx, s, wh, bh, wxt, bx):
    return pl.pallas_call(
        _layer_body,
        grid=(NP // LB,),
        in_specs=[
            pl.BlockSpec((LB, D), lambda i: (i, 0)),
            pl.BlockSpec((LB, 4), lambda i: (i, 0)),
            pl.BlockSpec((LB, ROWW), lambda i: (i, 0)),
            pl.BlockSpec((D, D), lambda i: (0, 0)),
            pl.BlockSpec((1, D), lambda i: (0, 0)),
            pl.BlockSpec((1, D), lambda i: (0, 0)),
            pl.BlockSpec((1, 1), lambda i: (0, 0)),
        ],
        out_specs=[
            pl.BlockSpec((LB, ROWW), lambda i: (i, 0)),
            pl.BlockSpec((LB, D), lambda i: (i, 0)),
            pl.BlockSpec((LB, 4), lambda i: (i, 0)),
        ],
        out_shape=[
            jax.ShapeDtypeStruct((NP, ROWW), jnp.float32),
            jax.ShapeDtypeStruct((NP, D), jnp.float32),
            jax.ShapeDtypeStruct((NP, 4), jnp.float32),
        ],
    )(h, xx, s, wh, bh, wxt, bx)


def _segsum_body(mq_hbm, colf_hbm, s_hbm, idx_v, rows_v, acc_v, sem):
    wid = lax.axis_index("s") * 2 + lax.axis_index("c")

    def batch(t, carry):
        nbase = wid * NPW + t * NB
        ebase = nbase * K
        pltpu.sync_copy(colf_hbm.at[pl.ds(ebase, NB * K)], idx_v)
        cps = []
        for c in range(NB * K // 128):
            cps.append(pltpu.async_copy(
                mq_hbm.at[idx_v.at[pl.ds(c * 128, 128)]],
                rows_v.at[pl.ds(c * 128, 128)], sem))
        for cp in cps:
            cp.wait()

        def node(n, carry2):
            r0 = n * K
            for c in range(ROWW // 16):
                sl = pl.ds(c * 16, 16)
                v = rows_v[r0, sl]
                for k in range(1, K):
                    v = v + rows_v[r0 + k, sl]
                acc_v[n, sl] = v
            return carry2

        lax.fori_loop(0, NB, node, 0)
        pltpu.sync_copy(acc_v, s_hbm.at[pl.ds(nbase, NB)])
        return carry

    lax.fori_loop(0, NT, batch, 0)


@functools.cache
def _build_segsum():
    return pl.kernel(
        _segsum_body,
        out_type=jax.ShapeDtypeStruct((NP, ROWW), jnp.float32),
        mesh=plsc.VectorSubcoreMesh(core_axis_name="c",
                                    subcore_axis_name="s"),
        compiler_params=pltpu.CompilerParams(use_tc_tiling_on_sc=False),
        scratch_types=[
            pltpu.VMEM((NB * K,), jnp.int32),
            pltpu.VMEM((NB * K, ROWW), jnp.float32),
            pltpu.VMEM((NB, ROWW), jnp.float32),
            pltpu.SemaphoreType.DMA,
        ],
    )


def _segsum(mq, colf):
    return _build_segsum()(mq, colf)


def _pool_body(hp_ref, sp_ref, out_ref):
    i = pl.program_id(0)

    @pl.when(i == 0)
    def _():
        out_ref[:] = jnp.zeros((1, D), jnp.float32)

    h = hp_ref[:] + sp_ref[:, 0:D]
    rows = lax.broadcasted_iota(jnp.int32, (LB, 1), 0) + i * LB
    hm = jnp.where(rows < NN, h, 0.0)
    out_ref[:] = out_ref[:] + jnp.sum(hm, axis=0, keepdims=True)


def _pool(h, s):
    return pl.pallas_call(
        _pool_body,
        grid=(NP // LB,),
        in_specs=[
            pl.BlockSpec((LB, D), lambda i: (i, 0)),
            pl.BlockSpec((LB, ROWW), lambda i: (i, 0)),
        ],
        out_specs=pl.BlockSpec((1, D), lambda i: (0, 0)),
        out_shape=jax.ShapeDtypeStruct((1, D), jnp.float32),
    )(h, s)


def _head_body(hsum_ref, wfc_ref, bfc_ref, wtt_ref, bt_ref, t_ref):
    hmean = hsum_ref[:] * (1.0 / NN)
    emb = jnp.maximum(
        jnp.dot(hmean, wfc_ref[:], preferred_element_type=jnp.float32)
        + bfc_ref[:], 0.0)
    for c in range(3):
        t_ref[0:1, c:c + 1] = (
            jnp.sum(emb * wtt_ref[c:c + 1, :], axis=1, keepdims=True)
            + bt_ref[0:1, c:c + 1])
    t_ref[0:1, 3:4] = jnp.zeros((1, 1), jnp.float32)


def _head(hsum, wfc, bfc, wtt, bt):
    return pl.pallas_call(
        _head_body,
        grid=(1,),
        in_specs=[
            pl.BlockSpec((1, D), lambda i: (0, 0)),
            pl.BlockSpec((D, D), lambda i: (0, 0)),
            pl.BlockSpec((1, D), lambda i: (0, 0)),
            pl.BlockSpec((3, D), lambda i: (0, 0)),
            pl.BlockSpec((1, 3), lambda i: (0, 0)),
        ],
        out_specs=pl.BlockSpec((1, 4), lambda i: (0, 0)),
        out_shape=jax.ShapeDtypeStruct((1, 4), jnp.float32),
    )(hsum, wfc, bfc, wtt, bt)


def _finish_body(xxp_ref, sp_ref, t_ref, out_ref):
    sw = sp_ref[:, 131:132]
    sxw = sp_ref[:, 128:131]
    xyz = xxp_ref[:, 0:3]
    xyz = xyz + xyz * sw - sxw
    out_ref[:] = xyz + t_ref[0:1, 0:3]


def _finish(xx, s, t):
    return pl.pallas_call(
        _finish_body,
        grid=(NP // LB,),
        in_specs=[
            pl.BlockSpec((LB, 4), lambda i: (i, 0)),
            pl.BlockSpec((LB, ROWW), lambda i: (i, 0)),
            pl.BlockSpec((1, 4), lambda i: (0, 0)),
        ],
        out_specs=pl.BlockSpec((LB, 3), lambda i: (i, 0)),
        out_shape=jax.ShapeDtypeStruct((NP, 3), jnp.float32),
    )(xx, s, t)


def kernel(x, pos, Wh0, bh0, Wx0, bx0, Wh1, bh1, Wx1, bx1, Wh2, bh2, Wx2,
           bx2, Wfc, bfc, Wt, bt):
    pos_p = jnp.pad(pos, ((0, NP - NN), (0, 0)),
                    constant_values=PADV)
    col = _knn(pos_p, pos_p.T)
    colf = col.reshape(-1)
    h = jnp.pad(x, ((0, NP - NN), (0, 0)))
    xx = jnp.pad(pos, ((0, NP - NN), (0, 1)))
    s = jnp.zeros((NP, ROWW), jnp.float32)
    for wh, bh, wx, bx in ((Wh0, bh0, Wx0, bx0), (Wh1, bh1, Wx1, bx1),
                           (Wh2, bh2, Wx2, bx2)):
        mq, h, xx = _layer(h, xx, s, wh, bh.reshape(1, D),
                           wx.reshape(1, D), bx.reshape(1, 1))
        s = _segsum(mq, colf)
    hsum = _pool(h, s)
    t = _head(hsum, Wfc, bfc.reshape(1, D), Wt.T, bt.reshape(1, 3))
    out = _finish(xx, s, t)
    return out[:NN]


# speedup vs baseline: 1.9017x; 1.0004x over previous
import functools

import jax
import jax.numpy as jnp
from jax import lax
from jax.experimental import pallas as pl
from jax.experimental.pallas import tpu as pltpu
from jax.experimental.pallas import tpu_sc as plsc

NN = 10000
K = 6
D = 128
RB = 256
CT = 512
LB = 512
NP = 10240
PADV = 1.0e18
ROWW = 144
NW = 32
NPW = NP // NW
NB = 64
NT = NPW // NB

_BIG_F = 3.0e38
_BIG_I = (1 << 30)


def _top6(vals, ids):
    out_v, out_i = [], []
    for _ in range(K):
        m = jnp.min(vals, axis=1, keepdims=True)
        idx = jnp.min(jnp.where(vals == m, ids, _BIG_I), axis=1,
                      keepdims=True)
        out_v.append(m)
        out_i.append(idx)
        vals = jnp.where(ids == idx, _BIG_F, vals)
    return out_v, out_i


def _knn_body(pos_r_ref, pos_t_ref, col_ref, bv_ref, bi_ref):
    i = pl.program_id(0)
    c = pl.program_id(1)

    @pl.when(c == 0)
    def _():
        bv_ref[:] = jnp.full((RB, 8), _BIG_F, jnp.float32)
        bi_ref[:] = jnp.zeros((RB, 8), jnp.int32)

    pr = pos_r_ref[:]
    pt = pos_t_ref[:]
    dot = jnp.dot(pr, pt, preferred_element_type=jnp.float32)
    sqr = (pr[:, 0:1] * pr[:, 0:1] + pr[:, 1:2] * pr[:, 1:2]) \
        + pr[:, 2:3] * pr[:, 2:3]
    sqc = (pt[0:1, :] * pt[0:1, :] + pt[1:2, :] * pt[1:2, :]) \
        + pt[2:3, :] * pt[2:3, :]
    d2 = (sqr + sqc) - 2.0 * dot
    colids = lax.broadcasted_iota(jnp.int32, (RB, CT), 1) + c * CT
    row_ids = lax.broadcasted_iota(jnp.int32, (RB, 1), 0) + i * RB
    d2 = jnp.where(colids == row_ids, _BIG_F, d2)
    tv, ti = _top6(d2, colids)

    lv, li = [], []
    for k in range(K):
        a_v = bv_ref[:, k:k + 1]
        a_i = bi_ref[:, k:k + 1]
        b_v = tv[K - 1 - k]
        b_i = ti[K - 1 - k]
        sel = a_v <= b_v
        lv.append(jnp.where(sel, a_v, b_v))
        li.append(jnp.where(sel, a_i, b_i))
    for r in range(K):
        for p, q in ((0, 1), (2, 3), (4, 5)) if r % 2 == 0 else \
                ((1, 2), (3, 4)):
            sel = lv[p] <= lv[q]
            pv = jnp.where(sel, lv[p], lv[q])
            qv = jnp.where(sel, lv[q], lv[p])
            pi = jnp.where(sel, li[p], li[q])
            qi = jnp.where(sel, li[q], li[p])
            lv[p], lv[q], li[p], li[q] = pv, qv, pi, qi
    for k in range(K):
        bv_ref[:, k:k + 1] = lv[k]
        bi_ref[:, k:k + 1] = li[k]

    @pl.when(c == NP // CT - 1)
    def _():
        for k in range(K):
            col_ref[:, k:k + 1] = li[k]


def _knn(pos_p, pos_t):
    return pl.pallas_call(
        _knn_body,
        grid=(NP // RB, NP // CT),
        in_specs=[
            pl.BlockSpec((RB, 3), lambda i, c: (i, 0)),
            pl.BlockSpec((3, CT), lambda i, c: (0, c)),
        ],
        out_specs=pl.BlockSpec((RB, K), lambda i, c: (i, 0)),
        out_shape=jax.ShapeDtypeStruct((NP, K), jnp.int32),
        scratch_shapes=[
            pltpu.VMEM((RB, 8), jnp.float32),
            pltpu.VMEM((RB, 8), jnp.int32),
        ],
        compiler_params=pltpu.CompilerParams(
            dimension_semantics=("parallel", "arbitrary")),
    )(pos_p, pos_t)


def _layer_body(hp_ref, xxp_ref, sp_ref, wh_ref, bh_ref, wx_ref, bx_ref,
                mq_ref, hn_ref, xxn_ref):
    h = hp_ref[:] + sp_ref[:, 0:D]
    sw = sp_ref[:, 131:132]
    sxw = sp_ref[:, 128:131]
    xyz = xxp_ref[:, 0:3]
    xyz = xyz + xyz * sw - sxw
    m = jnp.maximum(
        jnp.dot(h, wh_ref[:], preferred_element_type=jnp.float32)
        + bh_ref[:], 0.0)
    w = jnp.maximum(
        jnp.dot(h, wx_ref[:], preferred_element_type=jnp.float32)
        + bx_ref[:], 0.0)
    mq_ref[:, 0:D] = m
    mq_ref[:, 128:131] = xyz * w
    mq_ref[:, 131:132] = w
    mq_ref[:, 132:ROWW] = jnp.zeros((LB, ROWW - 132), jnp.float32)
    hn_ref[:] = h
    xxn_ref[:, 0:3] = xyz
    xxn_ref[:, 3:4] = jnp.zeros((LB, 1), jnp.float32)


def _layer(h, xx, s, wh, bh, wxt, bx):
    return pl.pallas_call(
        _layer_body,
        grid=(NP // LB,),
        in_specs=[
            pl.BlockSpec((LB, D), lambda i: (i, 0)),
            pl.BlockSpec((LB, 4), lambda i: (i, 0)),
            pl.BlockSpec((LB, ROWW), lambda i: (i, 0)),
            pl.BlockSpec((D, D), lambda i: (0, 0)),
            pl.BlockSpec((1, D), lambda i: (0, 0)),
            pl.BlockSpec((D, 1), lambda i: (0, 0)),
            pl.BlockSpec((1, 1), lambda i: (0, 0)),
        ],
        out_specs=[
            pl.BlockSpec((LB, ROWW), lambda i: (i, 0)),
            pl.BlockSpec((LB, D), lambda i: (i, 0)),
            pl.BlockSpec((LB, 4), lambda i: (i, 0)),
        ],
        out_shape=[
            jax.ShapeDtypeStruct((NP, ROWW), jnp.float32),
            jax.ShapeDtypeStruct((NP, D), jnp.float32),
            jax.ShapeDtypeStruct((NP, 4), jnp.float32),
        ],
    )(h, xx, s, wh, bh, wxt, bx)


def _segsum_body(mq_hbm, colf_hbm, s_hbm, idx_v, rows_v, acc_v, sem):
    wid = lax.axis_index("s") * 2 + lax.axis_index("c")

    def batch(t, carry):
        nbase = wid * NPW + t * NB
        ebase = nbase * K
        pltpu.sync_copy(colf_hbm.at[pl.ds(ebase, NB * K)], idx_v)
        cps = []
        for c in range(NB * K // 128):
            cps.append(pltpu.async_copy(
                mq_hbm.at[idx_v.at[pl.ds(c * 128, 128)]],
                rows_v.at[pl.ds(c * 128, 128)], sem))
        for cp in cps:
            cp.wait()

        def node(n, carry2):
            r0 = n * K
            for c in range(ROWW // 16):
                sl = pl.ds(c * 16, 16)
                v = rows_v[r0, sl]
                for k in range(1, K):
                    v = v + rows_v[r0 + k, sl]
                acc_v[n, sl] = v
            return carry2

        lax.fori_loop(0, NB, node, 0)
        pltpu.sync_copy(acc_v, s_hbm.at[pl.ds(nbase, NB)])
        return carry

    lax.fori_loop(0, NT, batch, 0)


@functools.cache
def _build_segsum():
    return pl.kernel(
        _segsum_body,
        out_type=jax.ShapeDtypeStruct((NP, ROWW), jnp.float32),
        mesh=plsc.VectorSubcoreMesh(core_axis_name="c",
                                    subcore_axis_name="s"),
        compiler_params=pltpu.CompilerParams(use_tc_tiling_on_sc=False),
        scratch_types=[
            pltpu.VMEM((NB * K,), jnp.int32),
            pltpu.VMEM((NB * K, ROWW), jnp.float32),
            pltpu.VMEM((NB, ROWW), jnp.float32),
            pltpu.SemaphoreType.DMA,
        ],
    )


def _segsum(mq, colf):
    return _build_segsum()(mq, colf)


def _pool_body(hp_ref, sp_ref, out_ref):
    i = pl.program_id(0)

    @pl.when(i == 0)
    def _():
        out_ref[:] = jnp.zeros((1, D), jnp.float32)

    h = hp_ref[:] + sp_ref[:, 0:D]
    rows = lax.broadcasted_iota(jnp.int32, (LB, 1), 0) + i * LB
    hm = jnp.where(rows < NN, h, 0.0)
    out_ref[:] = out_ref[:] + jnp.sum(hm, axis=0, keepdims=True)


def _pool(h, s):
    return pl.pallas_call(
        _pool_body,
        grid=(NP // LB,),
        in_specs=[
            pl.BlockSpec((LB, D), lambda i: (i, 0)),
            pl.BlockSpec((LB, ROWW), lambda i: (i, 0)),
        ],
        out_specs=pl.BlockSpec((1, D), lambda i: (0, 0)),
        out_shape=jax.ShapeDtypeStruct((1, D), jnp.float32),
    )(h, s)


def _head_body(hsum_ref, wfc_ref, bfc_ref, wt_ref, bt_ref, t_ref):
    hmean = hsum_ref[:] * (1.0 / NN)
    emb = jnp.maximum(
        jnp.dot(hmean, wfc_ref[:], preferred_element_type=jnp.float32)
        + bfc_ref[:], 0.0)
    t_ref[0:1, 0:3] = (
        jnp.dot(emb, wt_ref[:], preferred_element_type=jnp.float32)
        + bt_ref[:])
    t_ref[0:1, 3:4] = jnp.zeros((1, 1), jnp.float32)


def _head(hsum, wfc, bfc, wt, bt):
    return pl.pallas_call(
        _head_body,
        grid=(1,),
        in_specs=[
            pl.BlockSpec((1, D), lambda i: (0, 0)),
            pl.BlockSpec((D, D), lambda i: (0, 0)),
            pl.BlockSpec((1, D), lambda i: (0, 0)),
            pl.BlockSpec((D, 3), lambda i: (0, 0)),
            pl.BlockSpec((1, 3), lambda i: (0, 0)),
        ],
        out_specs=pl.BlockSpec((1, 4), lambda i: (0, 0)),
        out_shape=jax.ShapeDtypeStruct((1, 4), jnp.float32),
    )(hsum, wfc, bfc, wt, bt)


def _finish_body(xxp_ref, sp_ref, t_ref, out_ref):
    sw = sp_ref[:, 131:132]
    sxw = sp_ref[:, 128:131]
    xyz = xxp_ref[:, 0:3]
    xyz = xyz + xyz * sw - sxw
    out_ref[:] = xyz + t_ref[0:1, 0:3]


def _finish(xx, s, t):
    return pl.pallas_call(
        _finish_body,
        grid=(NP // LB,),
        in_specs=[
            pl.BlockSpec((LB, 4), lambda i: (i, 0)),
            pl.BlockSpec((LB, ROWW), lambda i: (i, 0)),
            pl.BlockSpec((1, 4), lambda i: (0, 0)),
        ],
        out_specs=pl.BlockSpec((LB, 3), lambda i: (i, 0)),
        out_shape=jax.ShapeDtypeStruct((NP, 3), jnp.float32),
    )(xx, s, t)


def kernel(x, pos, Wh0, bh0, Wx0, bx0, Wh1, bh1, Wx1, bx1, Wh2, bh2, Wx2,
           bx2, Wfc, bfc, Wt, bt):
    pos_p = jnp.pad(pos, ((0, NP - NN), (0, 0)),
                    constant_values=PADV)
    col = _knn(pos_p, pos_p.T)
    colf = col.reshape(-1)
    h = jnp.pad(x, ((0, NP - NN), (0, 0)))
    xx = jnp.pad(pos, ((0, NP - NN), (0, 1)))
    s = jnp.zeros((NP, ROWW), jnp.float32)
    for wh, bh, wx, bx in ((Wh0, bh0, Wx0, bx0), (Wh1, bh1, Wx1, bx1),
                           (Wh2, bh2, Wx2, bx2)):
        mq, h, xx = _layer(h, xx, s, wh, bh.reshape(1, D),
                           wx, bx.reshape(1, 1))
        s = _segsum(mq, colf)
    hsum = _pool(h, s)
    t = _head(hsum, Wfc, bfc.reshape(1, D), Wt, bt.reshape(1, 3))
    out = _finish(xx, s, t)
    return out[:NN]


# speedup vs baseline: 2.8278x; 1.4870x over previous
import functools

import jax
import jax.numpy as jnp
from jax import lax
from jax.experimental import pallas as pl
from jax.experimental.pallas import tpu as pltpu
from jax.experimental.pallas import tpu_sc as plsc

NN = 10000
K = 6
D = 128
RB = 256
CT = 1024
LB = 512
NP = 10240
PADV = 1.0e18
ROWW = 144
NW = 32
NPW = NP // NW
NB = 64
NT = NPW // NB

_BIG_F = 3.0e38
_BIG_I = (1 << 30)


def _top6(vals, ids):
    out_v, out_i = [], []
    for _ in range(K):
        m = jnp.min(vals, axis=1, keepdims=True)
        idx = jnp.min(jnp.where(vals == m, ids, _BIG_I), axis=1,
                      keepdims=True)
        out_v.append(m)
        out_i.append(idx)
        vals = jnp.where(ids == idx, _BIG_F, vals)
    return out_v, out_i


def _knn_body(pos_r_ref, pos_t_ref, col_ref, bv_ref, bi_ref):
    i = pl.program_id(0)
    c = pl.program_id(1)

    @pl.when(c == 0)
    def _():
        bv_ref[:] = jnp.full((RB, 8), _BIG_F, jnp.float32)
        bi_ref[:] = jnp.zeros((RB, 8), jnp.int32)

    pr = pos_r_ref[:]
    pt = pos_t_ref[:]
    dot = jnp.dot(pr, pt, preferred_element_type=jnp.float32)
    sqr = (pr[:, 0:1] * pr[:, 0:1] + pr[:, 1:2] * pr[:, 1:2]) \
        + pr[:, 2:3] * pr[:, 2:3]
    sqc = (pt[0:1, :] * pt[0:1, :] + pt[1:2, :] * pt[1:2, :]) \
        + pt[2:3, :] * pt[2:3, :]
    d2 = (sqr + sqc) - 2.0 * dot
    colids = lax.broadcasted_iota(jnp.int32, (RB, CT), 1) + c * CT
    row_ids = lax.broadcasted_iota(jnp.int32, (RB, 1), 0) + i * RB
    d2 = jnp.where(colids == row_ids, _BIG_F, d2)
    tv, ti = _top6(d2, colids)

    lv, li = [], []
    for k in range(K):
        a_v = bv_ref[:, k:k + 1]
        a_i = bi_ref[:, k:k + 1]
        b_v = tv[K - 1 - k]
        b_i = ti[K - 1 - k]
        sel = a_v <= b_v
        lv.append(jnp.where(sel, a_v, b_v))
        li.append(jnp.where(sel, a_i, b_i))
    for r in range(K):
        for p, q in ((0, 1), (2, 3), (4, 5)) if r % 2 == 0 else \
                ((1, 2), (3, 4)):
            sel = lv[p] <= lv[q]
            pv = jnp.where(sel, lv[p], lv[q])
            qv = jnp.where(sel, lv[q], lv[p])
            pi = jnp.where(sel, li[p], li[q])
            qi = jnp.where(sel, li[q], li[p])
            lv[p], lv[q], li[p], li[q] = pv, qv, pi, qi
    for k in range(K):
        bv_ref[:, k:k + 1] = lv[k]
        bi_ref[:, k:k + 1] = li[k]

    @pl.when(c == NP // CT - 1)
    def _():
        for k in range(K):
            col_ref[:, k:k + 1] = li[k]


def _knn(pos_p, pos_t):
    return pl.pallas_call(
        _knn_body,
        grid=(NP // RB, NP // CT),
        in_specs=[
            pl.BlockSpec((RB, 3), lambda i, c: (i, 0)),
            pl.BlockSpec((3, CT), lambda i, c: (0, c)),
        ],
        out_specs=pl.BlockSpec((RB, K), lambda i, c: (i, 0)),
        out_shape=jax.ShapeDtypeStruct((NP, K), jnp.int32),
        scratch_shapes=[
            pltpu.VMEM((RB, 8), jnp.float32),
            pltpu.VMEM((RB, 8), jnp.int32),
        ],
        compiler_params=pltpu.CompilerParams(
            dimension_semantics=("parallel", "arbitrary")),
    )(pos_p, pos_t)


def _layer_body(hp_ref, xxp_ref, sp_ref, wh_ref, bh_ref, wx_ref, bx_ref,
                mq_ref, hn_ref, xxn_ref):
    h = hp_ref[:] + sp_ref[:, 0:D]
    sw = sp_ref[:, 131:132]
    sxw = sp_ref[:, 128:131]
    xyz = xxp_ref[:, 0:3]
    xyz = xyz + xyz * sw - sxw
    m = jnp.maximum(
        jnp.dot(h, wh_ref[:], preferred_element_type=jnp.float32)
        + bh_ref[:], 0.0)
    w = jnp.maximum(
        jnp.dot(h, wx_ref[:], preferred_element_type=jnp.float32)
        + bx_ref[:], 0.0)
    mq_ref[:, 0:D] = m
    mq_ref[:, 128:131] = xyz * w
    mq_ref[:, 131:132] = w
    mq_ref[:, 132:ROWW] = jnp.zeros((LB, ROWW - 132), jnp.float32)
    hn_ref[:] = h
    xxn_ref[:, 0:3] = xyz
    xxn_ref[:, 3:4] = jnp.zeros((LB, 1), jnp.float32)


def _layer(h, xx, s, wh, bh, wxt, bx):
    return pl.pallas_call(
        _layer_body,
        grid=(NP // LB,),
        in_specs=[
            pl.BlockSpec((LB, D), lambda i: (i, 0)),
            pl.BlockSpec((LB, 4), lambda i: (i, 0)),
            pl.BlockSpec((LB, ROWW), lambda i: (i, 0)),
            pl.BlockSpec((D, D), lambda i: (0, 0)),
            pl.BlockSpec((1, D), lambda i: (0, 0)),
            pl.BlockSpec((D, 1), lambda i: (0, 0)),
            pl.BlockSpec((1, 1), lambda i: (0, 0)),
        ],
        out_specs=[
            pl.BlockSpec((LB, ROWW), lambda i: (i, 0)),
            pl.BlockSpec((LB, D), lambda i: (i, 0)),
            pl.BlockSpec((LB, 4), lambda i: (i, 0)),
        ],
        out_shape=[
            jax.ShapeDtypeStruct((NP, ROWW), jnp.float32),
            jax.ShapeDtypeStruct((NP, D), jnp.float32),
            jax.ShapeDtypeStruct((NP, 4), jnp.float32),
        ],
    )(h, xx, s, wh, bh, wxt, bx)


def _segsum_body(mq_hbm, colf_hbm, s_hbm, idx_v, rows_v, acc_v, sem):
    wid = lax.axis_index("s") * 2 + lax.axis_index("c")

    def batch(t, carry):
        nbase = wid * NPW + t * NB
        ebase = nbase * K
        pltpu.sync_copy(colf_hbm.at[pl.ds(ebase, NB * K)], idx_v)
        cps = []
        for c in range(NB * K // 128):
            cps.append(pltpu.async_copy(
                mq_hbm.at[idx_v.at[pl.ds(c * 128, 128)]],
                rows_v.at[pl.ds(c * 128, 128)], sem))
        for cp in cps:
            cp.wait()

        def node(n, carry2):
            r0 = n * K
            for c in range(ROWW // 16):
                sl = pl.ds(c * 16, 16)
                v = rows_v[r0, sl]
                for k in range(1, K):
                    v = v + rows_v[r0 + k, sl]
                acc_v[n, sl] = v
            return carry2

        lax.fori_loop(0, NB, node, 0)
        pltpu.sync_copy(acc_v, s_hbm.at[pl.ds(nbase, NB)])
        return carry

    lax.fori_loop(0, NT, batch, 0)


@functools.cache
def _build_segsum():
    return pl.kernel(
        _segsum_body,
        out_type=jax.ShapeDtypeStruct((NP, ROWW), jnp.float32),
        mesh=plsc.VectorSubcoreMesh(core_axis_name="c",
                                    subcore_axis_name="s"),
        compiler_params=pltpu.CompilerParams(use_tc_tiling_on_sc=False),
        scratch_types=[
            pltpu.VMEM((NB * K,), jnp.int32),
            pltpu.VMEM((NB * K, ROWW), jnp.float32),
            pltpu.VMEM((NB, ROWW), jnp.float32),
            pltpu.SemaphoreType.DMA,
        ],
    )


def _segsum(mq, colf):
    return _build_segsum()(mq, colf)


def _pool_body(hp_ref, sp_ref, out_ref):
    i = pl.program_id(0)

    @pl.when(i == 0)
    def _():
        out_ref[:] = jnp.zeros((1, D), jnp.float32)

    h = hp_ref[:] + sp_ref[:, 0:D]
    rows = lax.broadcasted_iota(jnp.int32, (LB, 1), 0) + i * LB
    hm = jnp.where(rows < NN, h, 0.0)
    out_ref[:] = out_ref[:] + jnp.sum(hm, axis=0, keepdims=True)


def _pool(h, s):
    return pl.pallas_call(
        _pool_body,
        grid=(NP // LB,),
        in_specs=[
            pl.BlockSpec((LB, D), lambda i: (i, 0)),
            pl.BlockSpec((LB, ROWW), lambda i: (i, 0)),
        ],
        out_specs=pl.BlockSpec((1, D), lambda i: (0, 0)),
        out_shape=jax.ShapeDtypeStruct((1, D), jnp.float32),
    )(h, s)


def _head_body(hsum_ref, wfc_ref, bfc_ref, wt_ref, bt_ref, t_ref):
    hmean = hsum_ref[:] * (1.0 / NN)
    emb = jnp.maximum(
        jnp.dot(hmean, wfc_ref[:], preferred_element_type=jnp.float32)
        + bfc_ref[:], 0.0)
    t_ref[0:1, 0:3] = (
        jnp.dot(emb, wt_ref[:], preferred_element_type=jnp.float32)
        + bt_ref[:])
    t_ref[0:1, 3:4] = jnp.zeros((1, 1), jnp.float32)


def _head(hsum, wfc, bfc, wt, bt):
    return pl.pallas_call(
        _head_body,
        grid=(1,),
        in_specs=[
            pl.BlockSpec((1, D), lambda i: (0, 0)),
            pl.BlockSpec((D, D), lambda i: (0, 0)),
            pl.BlockSpec((1, D), lambda i: (0, 0)),
            pl.BlockSpec((D, 3), lambda i: (0, 0)),
            pl.BlockSpec((1, 3), lambda i: (0, 0)),
        ],
        out_specs=pl.BlockSpec((1, 4), lambda i: (0, 0)),
        out_shape=jax.ShapeDtypeStruct((1, 4), jnp.float32),
    )(hsum, wfc, bfc, wt, bt)


def _finish_body(xxp_ref, sp_ref, t_ref, out_ref):
    sw = sp_ref[:, 131:132]
    sxw = sp_ref[:, 128:131]
    xyz = xxp_ref[:, 0:3]
    xyz = xyz + xyz * sw - sxw
    out_ref[:] = xyz + t_ref[0:1, 0:3]


def _finish(xx, s, t):
    return pl.pallas_call(
        _finish_body,
        grid=(NP // LB,),
        in_specs=[
            pl.BlockSpec((LB, 4), lambda i: (i, 0)),
            pl.BlockSpec((LB, ROWW), lambda i: (i, 0)),
            pl.BlockSpec((1, 4), lambda i: (0, 0)),
        ],
        out_specs=pl.BlockSpec((LB, 3), lambda i: (i, 0)),
        out_shape=jax.ShapeDtypeStruct((NP, 3), jnp.float32),
    )(xx, s, t)


def kernel(x, pos, Wh0, bh0, Wx0, bx0, Wh1, bh1, Wx1, bx1, Wh2, bh2, Wx2,
           bx2, Wfc, bfc, Wt, bt):
    pos_p = jnp.pad(pos, ((0, NP - NN), (0, 0)),
                    constant_values=PADV)
    col = _knn(pos_p, pos_p.T)
    colf = col.reshape(-1)
    h = jnp.pad(x, ((0, NP - NN), (0, 0)))
    xx = jnp.pad(pos, ((0, NP - NN), (0, 1)))
    s = jnp.zeros((NP, ROWW), jnp.float32)
    for wh, bh, wx, bx in ((Wh0, bh0, Wx0, bx0), (Wh1, bh1, Wx1, bx1),
                           (Wh2, bh2, Wx2, bx2)):
        mq, h, xx = _layer(h, xx, s, wh, bh.reshape(1, D),
                           wx, bx.reshape(1, 1))
        s = _segsum(mq, colf)
    hsum = _pool(h, s)
    t = _head(hsum, Wfc, bfc.reshape(1, D), Wt, bt.reshape(1, 3))
    out = _finish(xx, s, t)
    return out[:NN]


# speedup vs baseline: 5.1839x; 1.8332x over previous
import functools

import jax
import jax.numpy as jnp
from jax import lax
from jax.experimental import pallas as pl
from jax.experimental.pallas import tpu as pltpu
from jax.experimental.pallas import tpu_sc as plsc

NN = 10000
K = 6
D = 128
RB = 256
CT = 2048
LB = 512
NP = 10240
PADV = 1.0e18
ROWW = 144
NW = 32
NPW = NP // NW
NB = 64
NT = NPW // NB

_BIG_F = 3.0e38
_BIG_I = (1 << 30)


def _top6(vals, ids):
    out_v, out_i = [], []
    for _ in range(K):
        m = jnp.min(vals, axis=1, keepdims=True)
        idx = jnp.min(jnp.where(vals == m, ids, _BIG_I), axis=1,
                      keepdims=True)
        out_v.append(m)
        out_i.append(idx)
        vals = jnp.where(ids == idx, _BIG_F, vals)
    return out_v, out_i


def _knn_body(pos_r_ref, pos_t_ref, col_ref, bv_ref, bi_ref):
    i = pl.program_id(0)
    c = pl.program_id(1)

    @pl.when(c == 0)
    def _():
        bv_ref[:] = jnp.full((RB, 8), _BIG_F, jnp.float32)
        bi_ref[:] = jnp.zeros((RB, 8), jnp.int32)

    pr = pos_r_ref[:]
    pt = pos_t_ref[:]
    dot = jnp.dot(pr, pt, preferred_element_type=jnp.float32)
    sqr = (pr[:, 0:1] * pr[:, 0:1] + pr[:, 1:2] * pr[:, 1:2]) \
        + pr[:, 2:3] * pr[:, 2:3]
    sqc = (pt[0:1, :] * pt[0:1, :] + pt[1:2, :] * pt[1:2, :]) \
        + pt[2:3, :] * pt[2:3, :]
    d2 = (sqr + sqc) - 2.0 * dot
    colids = lax.broadcasted_iota(jnp.int32, (RB, CT), 1) + c * CT
    row_ids = lax.broadcasted_iota(jnp.int32, (RB, 1), 0) + i * RB
    d2 = jnp.where(colids == row_ids, _BIG_F, d2)
    tv, ti = _top6(d2, colids)

    lv, li = [], []
    for k in range(K):
        a_v = bv_ref[:, k:k + 1]
        a_i = bi_ref[:, k:k + 1]
        b_v = tv[K - 1 - k]
        b_i = ti[K - 1 - k]
        sel = a_v <= b_v
        lv.append(jnp.where(sel, a_v, b_v))
        li.append(jnp.where(sel, a_i, b_i))
    for r in range(K):
        for p, q in ((0, 1), (2, 3), (4, 5)) if r % 2 == 0 else \
                ((1, 2), (3, 4)):
            sel = lv[p] <= lv[q]
            pv = jnp.where(sel, lv[p], lv[q])
            qv = jnp.where(sel, lv[q], lv[p])
            pi = jnp.where(sel, li[p], li[q])
            qi = jnp.where(sel, li[q], li[p])
            lv[p], lv[q], li[p], li[q] = pv, qv, pi, qi
    for k in range(K):
        bv_ref[:, k:k + 1] = lv[k]
        bi_ref[:, k:k + 1] = li[k]

    @pl.when(c == NP // CT - 1)
    def _():
        for k in range(K):
            col_ref[:, k:k + 1] = li[k]


def _knn(pos_p, pos_t):
    return pl.pallas_call(
        _knn_body,
        grid=(NP // RB, NP // CT),
        in_specs=[
            pl.BlockSpec((RB, 3), lambda i, c: (i, 0)),
            pl.BlockSpec((3, CT), lambda i, c: (0, c)),
        ],
        out_specs=pl.BlockSpec((RB, K), lambda i, c: (i, 0)),
        out_shape=jax.ShapeDtypeStruct((NP, K), jnp.int32),
        scratch_shapes=[
            pltpu.VMEM((RB, 8), jnp.float32),
            pltpu.VMEM((RB, 8), jnp.int32),
        ],
        compiler_params=pltpu.CompilerParams(
            dimension_semantics=("parallel", "arbitrary")),
    )(pos_p, pos_t)


def _layer_body(hp_ref, xxp_ref, sp_ref, wh_ref, bh_ref, wx_ref, bx_ref,
                mq_ref, hn_ref, xxn_ref):
    h = hp_ref[:] + sp_ref[:, 0:D]
    sw = sp_ref[:, 131:132]
    sxw = sp_ref[:, 128:131]
    xyz = xxp_ref[:, 0:3]
    xyz = xyz + xyz * sw - sxw
    m = jnp.maximum(
        jnp.dot(h, wh_ref[:], preferred_element_type=jnp.float32)
        + bh_ref[:], 0.0)
    w = jnp.maximum(
        jnp.dot(h, wx_ref[:], preferred_element_type=jnp.float32)
        + bx_ref[:], 0.0)
    mq_ref[:, 0:D] = m
    mq_ref[:, 128:131] = xyz * w
    mq_ref[:, 131:132] = w
    mq_ref[:, 132:ROWW] = jnp.zeros((LB, ROWW - 132), jnp.float32)
    hn_ref[:] = h
    xxn_ref[:, 0:3] = xyz
    xxn_ref[:, 3:4] = jnp.zeros((LB, 1), jnp.float32)


def _layer(h, xx, s, wh, bh, wxt, bx):
    return pl.pallas_call(
        _layer_body,
        grid=(NP // LB,),
        in_specs=[
            pl.BlockSpec((LB, D), lambda i: (i, 0)),
            pl.BlockSpec((LB, 4), lambda i: (i, 0)),
            pl.BlockSpec((LB, ROWW), lambda i: (i, 0)),
            pl.BlockSpec((D, D), lambda i: (0, 0)),
            pl.BlockSpec((1, D), lambda i: (0, 0)),
            pl.BlockSpec((D, 1), lambda i: (0, 0)),
            pl.BlockSpec((1, 1), lambda i: (0, 0)),
        ],
        out_specs=[
            pl.BlockSpec((LB, ROWW), lambda i: (i, 0)),
            pl.BlockSpec((LB, D), lambda i: (i, 0)),
            pl.BlockSpec((LB, 4), lambda i: (i, 0)),
        ],
        out_shape=[
            jax.ShapeDtypeStruct((NP, ROWW), jnp.float32),
            jax.ShapeDtypeStruct((NP, D), jnp.float32),
            jax.ShapeDtypeStruct((NP, 4), jnp.float32),
        ],
    )(h, xx, s, wh, bh, wxt, bx)


def _segsum_body(mq_hbm, colf_hbm, s_hbm, idx_v, rows_v, acc_v, sem):
    wid = lax.axis_index("s") * 2 + lax.axis_index("c")

    def batch(t, carry):
        nbase = wid * NPW + t * NB
        ebase = nbase * K
        pltpu.sync_copy(colf_hbm.at[pl.ds(ebase, NB * K)], idx_v)
        cps = []
        for c in range(NB * K // 128):
            cps.append(pltpu.async_copy(
                mq_hbm.at[idx_v.at[pl.ds(c * 128, 128)]],
                rows_v.at[pl.ds(c * 128, 128)], sem))
        for cp in cps:
            cp.wait()

        def node(n, carry2):
            r0 = n * K
            for c in range(ROWW // 16):
                sl = pl.ds(c * 16, 16)
                v = rows_v[r0, sl]
                for k in range(1, K):
                    v = v + rows_v[r0 + k, sl]
                acc_v[n, sl] = v
            return carry2

        lax.fori_loop(0, NB, node, 0)
        pltpu.sync_copy(acc_v, s_hbm.at[pl.ds(nbase, NB)])
        return carry

    lax.fori_loop(0, NT, batch, 0)


@functools.cache
def _build_segsum():
    return pl.kernel(
        _segsum_body,
        out_type=jax.ShapeDtypeStruct((NP, ROWW), jnp.float32),
        mesh=plsc.VectorSubcoreMesh(core_axis_name="c",
                                    subcore_axis_name="s"),
        compiler_params=pltpu.CompilerParams(use_tc_tiling_on_sc=False),
        scratch_types=[
            pltpu.VMEM((NB * K,), jnp.int32),
            pltpu.VMEM((NB * K, ROWW), jnp.float32),
            pltpu.VMEM((NB, ROWW), jnp.float32),
            pltpu.SemaphoreType.DMA,
        ],
    )


def _segsum(mq, colf):
    return _build_segsum()(mq, colf)


def _pool_body(hp_ref, sp_ref, out_ref):
    i = pl.program_id(0)

    @pl.when(i == 0)
    def _():
        out_ref[:] = jnp.zeros((1, D), jnp.float32)

    h = hp_ref[:] + sp_ref[:, 0:D]
    rows = lax.broadcasted_iota(jnp.int32, (LB, 1), 0) + i * LB
    hm = jnp.where(rows < NN, h, 0.0)
    out_ref[:] = out_ref[:] + jnp.sum(hm, axis=0, keepdims=True)


def _pool(h, s):
    return pl.pallas_call(
        _pool_body,
        grid=(NP // LB,),
        in_specs=[
            pl.BlockSpec((LB, D), lambda i: (i, 0)),
            pl.BlockSpec((LB, ROWW), lambda i: (i, 0)),
        ],
        out_specs=pl.BlockSpec((1, D), lambda i: (0, 0)),
        out_shape=jax.ShapeDtypeStruct((1, D), jnp.float32),
    )(h, s)


def _head_body(hsum_ref, wfc_ref, bfc_ref, wt_ref, bt_ref, t_ref):
    hmean = hsum_ref[:] * (1.0 / NN)
    emb = jnp.maximum(
        jnp.dot(hmean, wfc_ref[:], preferred_element_type=jnp.float32)
        + bfc_ref[:], 0.0)
    t_ref[0:1, 0:3] = (
        jnp.dot(emb, wt_ref[:], preferred_element_type=jnp.float32)
        + bt_ref[:])
    t_ref[0:1, 3:4] = jnp.zeros((1, 1), jnp.float32)


def _head(hsum, wfc, bfc, wt, bt):
    return pl.pallas_call(
        _head_body,
        grid=(1,),
        in_specs=[
            pl.BlockSpec((1, D), lambda i: (0, 0)),
            pl.BlockSpec((D, D), lambda i: (0, 0)),
            pl.BlockSpec((1, D), lambda i: (0, 0)),
            pl.BlockSpec((D, 3), lambda i: (0, 0)),
            pl.BlockSpec((1, 3), lambda i: (0, 0)),
        ],
        out_specs=pl.BlockSpec((1, 4), lambda i: (0, 0)),
        out_shape=jax.ShapeDtypeStruct((1, 4), jnp.float32),
    )(hsum, wfc, bfc, wt, bt)


def _finish_body(xxp_ref, sp_ref, t_ref, out_ref):
    sw = sp_ref[:, 131:132]
    sxw = sp_ref[:, 128:131]
    xyz = xxp_ref[:, 0:3]
    xyz = xyz + xyz * sw - sxw
    out_ref[:] = xyz + t_ref[0:1, 0:3]


def _finish(xx, s, t):
    return pl.pallas_call(
        _finish_body,
        grid=(NP // LB,),
        in_specs=[
            pl.BlockSpec((LB, 4), lambda i: (i, 0)),
            pl.BlockSpec((LB, ROWW), lambda i: (i, 0)),
            pl.BlockSpec((1, 4), lambda i: (0, 0)),
        ],
        out_specs=pl.BlockSpec((LB, 3), lambda i: (i, 0)),
        out_shape=jax.ShapeDtypeStruct((NP, 3), jnp.float32),
    )(xx, s, t)


def kernel(x, pos, Wh0, bh0, Wx0, bx0, Wh1, bh1, Wx1, bx1, Wh2, bh2, Wx2,
           bx2, Wfc, bfc, Wt, bt):
    pos_p = jnp.pad(pos, ((0, NP - NN), (0, 0)),
                    constant_values=PADV)
    col = _knn(pos_p, pos_p.T)
    colf = col.reshape(-1)
    h = jnp.pad(x, ((0, NP - NN), (0, 0)))
    xx = jnp.pad(pos, ((0, NP - NN), (0, 1)))
    s = jnp.zeros((NP, ROWW), jnp.float32)
    for wh, bh, wx, bx in ((Wh0, bh0, Wx0, bx0), (Wh1, bh1, Wx1, bx1),
                           (Wh2, bh2, Wx2, bx2)):
        mq, h, xx = _layer(h, xx, s, wh, bh.reshape(1, D),
                           wx, bx.reshape(1, 1))
        s = _segsum(mq, colf)
    hsum = _pool(h, s)
    t = _head(hsum, Wfc, bfc.reshape(1, D), Wt, bt.reshape(1, 3))
    out = _finish(xx, s, t)
    return out[:NN]
